# baseline (speedup 1.0000x reference)
import functools

import jax
import jax.numpy as jnp
from jax import lax
from jax.experimental import pallas as pl
from jax.experimental.pallas import tpu as pltpu
from jax.experimental.pallas import tpu_sc as plsc

_GAMMA = 12.0
_D = 64
_B = 16384
_NC = 2
_NS = 16
_L = 16
_NW = _NC * _NS
_BPW = _B // _NW
_CH = 128
_NCH = _BPW // _CH
_GPC = _CH // _L

_mesh = plsc.VectorSubcoreMesh(core_axis_name="c", subcore_axis_name="s")


@functools.partial(
    pl.kernel,
    out_type=jax.ShapeDtypeStruct((_B,), jnp.float32),
    mesh=_mesh,
    scratch_types=[
        pltpu.VMEM((_BPW,), jnp.int32),
        pltpu.VMEM((_BPW,), jnp.int32),
        pltpu.VMEM((_BPW,), jnp.int32),
        pltpu.VMEM((_BPW,), jnp.int32),
        pltpu.VMEM((_BPW,), jnp.int32),
        pltpu.VMEM((_BPW,), jnp.int32),
        pltpu.VMEM((_CH, 2 * _D), jnp.float32),
        pltpu.VMEM((_CH, 2 * _D), jnp.float32),
        pltpu.VMEM((_CH, 2 * _D), jnp.float32),
        pltpu.VMEM((_CH, 2 * _D), jnp.float32),
        pltpu.VMEM((_CH, 2 * _D), jnp.float32),
        pltpu.VMEM((_CH, 2 * _D), jnp.float32),
        pltpu.VMEM((_BPW,), jnp.float32),
        pltpu.SemaphoreType.DMA,
        pltpu.SemaphoreType.DMA,
    ],
    compiler_params=pltpu.CompilerParams(
        needs_layout_passes=False, use_tc_tiling_on_sc=True),
)
def _kge_score(ent_hbm, relemb_hbm, head_hbm, rel_hbm, tail_hbm, out_hbm,
               idx_h, idx_r, idx_t, par_h, par_r, par_t,
               h0, h1, r0, r1, t0, t1, o_v, sem0, sem1):
    wid = lax.axis_index("s") * _NC + lax.axis_index("c")
    base = wid * _BPW

    pltpu.sync_copy(head_hbm.at[pl.ds(base, _BPW)], idx_h)
    pltpu.sync_copy(rel_hbm.at[pl.ds(base, _BPW)], idx_r)
    pltpu.sync_copy(tail_hbm.at[pl.ds(base, _BPW)], idx_t)

    def prep(v, _):
        sl = pl.ds(v * _L, _L)
        for idx, par in ((idx_h, par_h), (idx_r, par_r), (idx_t, par_t)):
            iv = idx[sl]
            par[sl] = (iv & 1) << 6
            idx[sl] = iv >> 1
        return _

    lax.fori_loop(0, _BPW // _L, prep, 0)

    bufs = ((h0, r0, t0, sem0), (h1, r1, t1, sem1))

    def fire(c, hb, rb, tb, sem):
        sl = pl.ds(c * _CH, _CH)
        copies = (
            pltpu.async_copy(ent_hbm.at[idx_h.at[sl]], hb, sem),
            pltpu.async_copy(relemb_hbm.at[idx_r.at[sl]], rb, sem),
            pltpu.async_copy(ent_hbm.at[idx_t.at[sl]], tb, sem),
        )
        return copies

    lanes = lax.iota(jnp.int32, _L)
    chunk_cols = [lanes + c * _L for c in range(_D // _L)]

    pend = fire(0, *bufs[0])

    for c in range(_NCH):
        for cp in pend:
            cp.wait()
        if c + 1 < _NCH:
            pend = fire(c + 1, *bufs[(c + 1) % 2])
        hb, rb, tb, _ = bufs[c % 2]

        def group(g, carry):
            psl = pl.ds(c * _CH + g * _L, _L)
            ph = par_h[psl]
            pr = par_r[psl]
            pt = par_t[psl]
            acc = jnp.zeros((_L,), jnp.float32)
            for i in range(_L):
                isel = jnp.full((_L,), i, jnp.int32)
                row = jnp.full((_L,), g * _L + i, jnp.int32)
                bh = ph.at[isel].get(mode="promise_in_bounds")
                br = pr.at[isel].get(mode="promise_in_bounds")
                bt = pt.at[isel].get(mode="promise_in_bounds")
                s = jnp.zeros((_L,), jnp.float32)
                for cc in chunk_cols:
                    hv = plsc.load_gather(hb, [row, bh + cc])
                    rv = plsc.load_gather(rb, [row, br + cc])
                    tv = plsc.load_gather(tb, [row, bt + cc])
                    diff = hv + rv - tv
                    s = s + diff * diff
                tot = lax.reduce_sum_p.bind(s, axes=(0,))
                acc = jnp.where(lanes == i, tot, acc)
            x = acc + 1e-12
            seed = plsc.bitcast(
                (plsc.bitcast(x, jnp.int32) >> 1) + 0x1FBD1DF5, jnp.float32)
            y = 0.5 * (seed + x / seed)
            y = 0.5 * (y + x / y)
            o_v[pl.ds(c * _CH + g * _L, _L)] = _GAMMA - y
            return carry

        lax.fori_loop(0, _GPC, group, 0)

    pltpu.sync_copy(o_v, out_hbm.at[pl.ds(base, _BPW)])


def kernel(entity_emb, relation_emb, head, rel, tail):
    ent2 = entity_emb.reshape(50000, 2 * _D)
    rel2 = relation_emb.reshape(500, 2 * _D)
    return _kge_score(ent2, rel2, head, rel, tail)
